# speedup vs baseline: 1.9692x; 1.9692x over previous
import functools

import jax
import jax.numpy as jnp
from jax import lax
from jax.experimental import pallas as pl
from jax.experimental.pallas import tpu as pltpu
from jax.experimental.pallas import tpu_sc as plsc

_V = 8192
_D = 8192
_NB = 8192
_NC = 2
_NS = 16
_NW = _NC * _NS
_BPW = _NB // _NW
_K = 4
_NCHUNK = _BPW // _K


@functools.partial(
    pl.kernel,
    out_type=jax.ShapeDtypeStruct((_NB, _D), jnp.float32),
    mesh=plsc.VectorSubcoreMesh(core_axis_name="c", subcore_axis_name="s"),
    scratch_types=[
        pltpu.VMEM((_NCHUNK, _K), jnp.int32),
        pltpu.VMEM((_K, _D), jnp.float32),
        pltpu.VMEM((_K, _D), jnp.float32),
        pltpu.SemaphoreType.DMA,
        pltpu.SemaphoreType.DMA,
        pltpu.SemaphoreType.DMA,
        pltpu.SemaphoreType.DMA,
    ],
)
def _gather_rows(x_hbm, table_hbm, out_hbm, idx_v, buf0, buf1, g0, g1, s0, s1):
    wid = lax.axis_index("s") * _NC + lax.axis_index("c")
    base = wid * _BPW
    pltpu.sync_copy(x_hbm.at[wid], idx_v)

    bufs = (buf0, buf1)
    gsems = (g0, g1)
    ssems = (s0, s1)

    def gather_start(cur, b):
        pltpu.async_copy(table_hbm.at[idx_v.at[cur]], bufs[b], gsems[b])

    def gather_wait(cur, b):
        pltpu.make_async_copy(table_hbm.at[idx_v.at[cur]], bufs[b], gsems[b]).wait()

    def scatter_start(cur, b):
        pltpu.async_copy(bufs[b], out_hbm.at[pl.ds(base + cur * _K, _K)], ssems[b])

    def scatter_wait(cur, b):
        pltpu.make_async_copy(
            bufs[b], out_hbm.at[pl.ds(base + cur * _K, _K)], ssems[b]
        ).wait()

    gather_start(0, 0)
    gather_start(1, 1)

    def body(i, carry):
        c = i * 2
        for b in range(2):
            cur = c + b
            gather_wait(cur, b)
            scatter_start(cur, b)
            scatter_wait(cur, b)
            nxt = cur + 2

            @pl.when(nxt < _NCHUNK)
            def _():
                gather_start(nxt, b)

        return carry

    lax.fori_loop(0, _NCHUNK // 2, body, 0)


def kernel(x, table):
    x3 = x.reshape(_NW, _NCHUNK, _K).astype(jnp.int32)
    out = _gather_rows(x3, table)
    return out.reshape(x.shape[0], x.shape[1], _D)
